# baseline (speedup 1.0000x reference)
import functools

import jax
import jax.numpy as jnp
from jax import lax
from jax.experimental import pallas as pl
from jax.experimental.pallas import tpu as pltpu
from jax.experimental.pallas import tpu_sc as plsc

B = 1024
LT = 50
LV = 49
D = 512
S = LV + 1 + LT
HF = LT * D

NC = 2
NS = 16
NW = NC * NS
NTOK = B * LT
PER_W = NTOK // NW
CH = 64
NCHUNK = PER_W // CH


def _sc_gather_body(ids_hbm, table_hbm, out_hbm, idx_v, rows_v, sem):
    wid = lax.axis_index("s") * NC + lax.axis_index("c")
    base = wid * PER_W

    def chunk(i, carry):
        off = pl.multiple_of(base + i * CH, CH)
        pltpu.sync_copy(ids_hbm.at[pl.ds(off, CH)], idx_v)
        pltpu.async_copy(table_hbm.at[idx_v], rows_v, sem).wait()
        pltpu.sync_copy(rows_v, out_hbm.at[pl.ds(off, CH)])
        return carry

    lax.fori_loop(0, NCHUNK, chunk, 0)


@jax.jit
def _sc_gather(ids_flat, table):
    mesh = plsc.VectorSubcoreMesh(core_axis_name="c", subcore_axis_name="s")
    fn = functools.partial(
        pl.kernel,
        mesh=mesh,
        out_type=jax.ShapeDtypeStruct((NTOK, D), jnp.float32),
        scratch_types=[
            pltpu.VMEM((CH,), jnp.int32),
            pltpu.VMEM((CH, D), jnp.float32),
            pltpu.SemaphoreType.DMA,
        ],
    )(_sc_gather_body)
    return fn(ids_flat, table)


BB = 16


def _ln(x, gamma, beta):
    mu = jnp.mean(x, axis=-1, keepdims=True)
    xc = x - mu
    var = jnp.mean(xc * xc, axis=-1, keepdims=True)
    return xc * lax.rsqrt(var + 1e-5) * gamma + beta


def _tc_vis_body(vis_ref, pos_ref, type_ref, cls_ref, gamma_ref, beta_ref,
                 out_ref):
    gamma = gamma_ref[...][None]
    beta = beta_ref[...][None]
    row0 = cls_ref[...] + type_ref[0:1] + pos_ref[0:1]
    y0 = _ln(row0[None], gamma, beta)
    vis = vis_ref[...].reshape(BB, LV, D)
    bvis = (pos_ref[1:LV + 1] + type_ref[0:1])[None]
    y = _ln(vis + bvis, gamma, beta)
    full = jnp.concatenate([jnp.broadcast_to(y0, (BB, 1, D)), y], axis=1)
    out_ref[...] = full.reshape(BB, HF)


def _tc_txt_body(full_ref, g_ref, pos_ref, type_ref, gamma_ref, beta_ref,
                 out_ref):
    del full_ref
    gamma = gamma_ref[...][None]
    beta = beta_ref[...][None]
    btxt = (pos_ref[...] + type_ref[1:2])[None]
    g = g_ref[...].reshape(BB, LT, D)
    out_ref[...] = _ln(g + btxt, gamma, beta).reshape(BB, HF)


@jax.jit
def _tc_vis(vis2d, pos50, type_table, cls2d, gamma2d, beta2d):
    return pl.pallas_call(
        _tc_vis_body,
        grid=(B // BB,),
        in_specs=[
            pl.BlockSpec((BB, LV * D), lambda i: (i, 0)),
            pl.BlockSpec((LV + 1, D), lambda i: (0, 0)),
            pl.BlockSpec((2, D), lambda i: (0, 0)),
            pl.BlockSpec((1, D), lambda i: (0, 0)),
            pl.BlockSpec((1, D), lambda i: (0, 0)),
            pl.BlockSpec((1, D), lambda i: (0, 0)),
        ],
        out_specs=pl.BlockSpec((BB, HF), lambda i: (i, 0)),
        out_shape=jax.ShapeDtypeStruct((B, 2 * HF), jnp.float32),
    )(vis2d, pos50, type_table, cls2d, gamma2d, beta2d)


@jax.jit
def _tc_txt(vis_out, g2d, pos_txt, type_table, gamma2d, beta2d):
    return pl.pallas_call(
        _tc_txt_body,
        grid=(B // BB,),
        in_specs=[
            pl.BlockSpec(memory_space=pl.ANY),
            pl.BlockSpec((BB, HF), lambda i: (i, 0)),
            pl.BlockSpec((LT, D), lambda i: (0, 0)),
            pl.BlockSpec((2, D), lambda i: (0, 0)),
            pl.BlockSpec((1, D), lambda i: (0, 0)),
            pl.BlockSpec((1, D), lambda i: (0, 0)),
        ],
        out_specs=pl.BlockSpec((BB, HF), lambda i: (i, 1)),
        out_shape=jax.ShapeDtypeStruct((B, 2 * HF), jnp.float32),
        input_output_aliases={0: 0},
    )(vis_out, g2d, pos_txt, type_table, gamma2d, beta2d)


def kernel(input_ids, visual_embeds, token_table, type_table,
           class_embedding, pos_table, ln_gamma, ln_beta):
    ids_flat = input_ids.reshape(-1).astype(jnp.int32)
    g = _sc_gather(ids_flat, token_table)
    g2d = g.reshape(B, HF)
    cls2d = class_embedding[None]
    gamma2d = ln_gamma[None]
    beta2d = ln_beta[None]
    vis_out = _tc_vis(visual_embeds.reshape(B, LV * D), pos_table[:LV + 1],
                      type_table, cls2d, gamma2d, beta2d)
    out = _tc_txt(vis_out, g2d, pos_table[LV + 1:S], type_table, gamma2d,
                  beta2d)
    return out.reshape(B, S, D)


# speedup vs baseline: 1.6078x; 1.6078x over previous
import functools

import jax
import jax.numpy as jnp
from jax import lax
from jax.experimental import pallas as pl
from jax.experimental.pallas import tpu as pltpu
from jax.experimental.pallas import tpu_sc as plsc

B = 1024
LT = 50
LV = 49
D = 512
S = LV + 1 + LT

NC = 2
NS = 16
NW = NC * NS

HB = B // 2
NTOK = HB * LT
PER_W = NTOK // NW
CH = 80
NCHUNK = PER_W // CH


def _sc_gather_body(ids_hbm, table_hbm, out_hbm, idx_v, rows_v, sem):
    wid = lax.axis_index("s") * NC + lax.axis_index("c")
    base = wid * PER_W

    def chunk(i, carry):
        off = pl.multiple_of(base + i * CH, 8)
        pltpu.sync_copy(ids_hbm.at[pl.ds(off, CH)], idx_v)
        pltpu.async_copy(table_hbm.at[idx_v], rows_v, sem).wait()
        pltpu.sync_copy(rows_v, out_hbm.at[pl.ds(off, CH)])
        return carry

    lax.fori_loop(0, NCHUNK, chunk, 0)


def _sc_gather(ids_flat, table):
    mesh = plsc.VectorSubcoreMesh(core_axis_name="c", subcore_axis_name="s")
    fn = functools.partial(
        pl.kernel,
        mesh=mesh,
        out_type=jax.ShapeDtypeStruct((NTOK, D), jnp.float32),
        scratch_types=[
            pltpu.VMEM((CH,), jnp.int32),
            pltpu.VMEM((CH, D), jnp.float32),
            pltpu.SemaphoreType.DMA,
        ],
    )(_sc_gather_body)
    return fn(ids_flat, table)


BB = 16


def _ln(x, gamma, beta):
    mu = jnp.mean(x, axis=-1, keepdims=True)
    xc = x - mu
    var = jnp.mean(xc * xc, axis=-1, keepdims=True)
    return xc * lax.rsqrt(var + 1e-5) * gamma + beta


def _tc_body(g_ref, vis_ref, pos_ref, type_ref, cls_ref, gamma_ref,
             beta_ref, out_ref):
    gamma = gamma_ref[...][None]
    beta = beta_ref[...][None]
    row0 = cls_ref[...] + type_ref[0:1] + pos_ref[0:1]
    y0 = _ln(row0[None], gamma, beta)
    out_ref[:, 0:1, :] = jnp.broadcast_to(y0, (BB, 1, D))
    bvis = (pos_ref[1:LV + 1] + type_ref[0:1])[None]
    out_ref[:, 1:LV + 1, :] = _ln(vis_ref[...] + bvis, gamma, beta)
    btxt = (pos_ref[LV + 1:S] + type_ref[1:2])[None]
    out_ref[:, LV + 1:S, :] = _ln(g_ref[...] + btxt, gamma, beta)


def _tc_aliased_body(full_ref, *rest):
    del full_ref
    _tc_body(*rest)


def _tc_half(off_blocks, aliased, operands):
    specs = [
        pl.BlockSpec((BB, LT, D), lambda i: (i, 0, 0)),
        pl.BlockSpec((BB, LV, D), lambda i, o=off_blocks: (i + o, 0, 0)),
        pl.BlockSpec((S, D), lambda i: (0, 0)),
        pl.BlockSpec((2, D), lambda i: (0, 0)),
        pl.BlockSpec((1, D), lambda i: (0, 0)),
        pl.BlockSpec((1, D), lambda i: (0, 0)),
        pl.BlockSpec((1, D), lambda i: (0, 0)),
    ]
    body = _tc_body
    alias_kw = {}
    if aliased:
        specs = [pl.BlockSpec(memory_space=pl.ANY)] + specs
        body = _tc_aliased_body
        alias_kw = {"input_output_aliases": {0: 0}}
    return pl.pallas_call(
        body,
        grid=(HB // BB,),
        in_specs=specs,
        out_specs=pl.BlockSpec((BB, S, D),
                               lambda i, o=off_blocks: (i + o, 0, 0)),
        out_shape=jax.ShapeDtypeStruct((B, S, D), jnp.float32),
        **alias_kw,
    )(*operands)


def kernel(input_ids, visual_embeds, token_table, type_table,
           class_embedding, pos_table, ln_gamma, ln_beta):
    ids = input_ids.reshape(-1).astype(jnp.int32)
    g_a = _sc_gather(ids[:NTOK], token_table).reshape(HB, LT, D)
    g_b = _sc_gather(ids[NTOK:], token_table).reshape(HB, LT, D)
    pos100 = pos_table[:S]
    cls2d = class_embedding[None]
    gamma2d = ln_gamma[None]
    beta2d = ln_beta[None]
    consts = (pos100, type_table, cls2d, gamma2d, beta2d)
    out_a = _tc_half(0, False, (g_a, visual_embeds) + consts)
    out_b = _tc_half(HB // BB, True, (out_a, g_b, visual_embeds) + consts)
    return out_b


# speedup vs baseline: 3.5433x; 2.2038x over previous
import functools

import jax
import jax.numpy as jnp
from jax import lax
from jax.experimental import pallas as pl
from jax.experimental.pallas import tpu as pltpu
from jax.experimental.pallas import tpu_sc as plsc

B = 1024
LT = 50
LV = 49
D = 512
S = LV + 1 + LT

NC = 2
NS = 16
NW = NC * NS
NTOK = B * LT
PER_W = NTOK // NW
CH = 64
NCHUNK = PER_W // CH


def _sc_gather_body(ids_hbm, table_hbm, out_hbm, idx_v, rows_v, sem):
    wid = lax.axis_index("s") * NC + lax.axis_index("c")
    base = wid * PER_W

    def chunk(i, carry):
        off = pl.multiple_of(base + i * CH, CH)
        pltpu.sync_copy(ids_hbm.at[pl.ds(off, CH)], idx_v)
        pltpu.async_copy(table_hbm.at[idx_v], rows_v, sem).wait()
        pltpu.sync_copy(rows_v, out_hbm.at[pl.ds(off, CH)])
        return carry

    lax.fori_loop(0, NCHUNK, chunk, 0)


def _sc_gather(ids_flat, table):
    mesh = plsc.VectorSubcoreMesh(core_axis_name="c", subcore_axis_name="s")
    fn = functools.partial(
        pl.kernel,
        mesh=mesh,
        out_type=jax.ShapeDtypeStruct((NTOK, D), jnp.float32),
        scratch_types=[
            pltpu.VMEM((CH,), jnp.int32),
            pltpu.VMEM((CH, D), jnp.float32),
            pltpu.SemaphoreType.DMA,
        ],
    )(_sc_gather_body)
    return fn(ids_flat, table)


BB = 16


def _ln(x, gamma, beta):
    mu = jnp.mean(x, axis=-1, keepdims=True)
    xc = x - mu
    var = jnp.mean(xc * xc, axis=-1, keepdims=True)
    return xc * lax.rsqrt(var + 1e-5) * gamma + beta


def _tc_vis_body(vis_ref, pos_ref, type_ref, cls_ref, gamma_ref, beta_ref,
                 out_ref):
    gamma = gamma_ref[...][None]
    beta = beta_ref[...][None]
    row0 = cls_ref[...] + type_ref[0:1] + pos_ref[0:1]
    y0 = _ln(row0[None], gamma, beta)
    out_ref[0:1] = jnp.broadcast_to(y0, (1, BB, D))
    bvis = (pos_ref[1:LV + 1] + type_ref[0:1])[:, None]
    out_ref[1:LV + 1] = _ln(vis_ref[...] + bvis, gamma, beta)


def _tc_txt_body(full_ref, g_ref, pos_ref, type_ref, gamma_ref, beta_ref,
                 out_ref):
    del full_ref
    gamma = gamma_ref[...][None]
    beta = beta_ref[...][None]
    btxt = (pos_ref[LV + 1:S] + type_ref[1:2])[:, None]
    out_ref[...] = _ln(g_ref[...] + btxt, gamma, beta)


def _tc_vis(vis_t, pos100, type_table, cls2d, gamma2d, beta2d):
    return pl.pallas_call(
        _tc_vis_body,
        grid=(B // BB,),
        in_specs=[
            pl.BlockSpec((LV, BB, D), lambda i: (0, i, 0)),
            pl.BlockSpec((S, D), lambda i: (0, 0)),
            pl.BlockSpec((2, D), lambda i: (0, 0)),
            pl.BlockSpec((1, D), lambda i: (0, 0)),
            pl.BlockSpec((1, D), lambda i: (0, 0)),
            pl.BlockSpec((1, D), lambda i: (0, 0)),
        ],
        out_specs=pl.BlockSpec((LV + 1, BB, D), lambda i: (0, i, 0)),
        out_shape=jax.ShapeDtypeStruct((S, B, D), jnp.float32),
    )(vis_t, pos100, type_table, cls2d, gamma2d, beta2d)


def _tc_txt(vis_out, g_t, pos100, type_table, gamma2d, beta2d):
    return pl.pallas_call(
        _tc_txt_body,
        grid=(B // BB,),
        in_specs=[
            pl.BlockSpec(memory_space=pl.ANY),
            pl.BlockSpec((LT, BB, D), lambda i: (0, i, 0)),
            pl.BlockSpec((S, D), lambda i: (0, 0)),
            pl.BlockSpec((2, D), lambda i: (0, 0)),
            pl.BlockSpec((1, D), lambda i: (0, 0)),
            pl.BlockSpec((1, D), lambda i: (0, 0)),
        ],
        out_specs=pl.BlockSpec((LT, BB, D), lambda i: (1, i, 0)),
        out_shape=jax.ShapeDtypeStruct((S, B, D), jnp.float32),
        input_output_aliases={0: 0},
    )(vis_out, g_t, pos100, type_table, gamma2d, beta2d)


def kernel(input_ids, visual_embeds, token_table, type_table,
           class_embedding, pos_table, ln_gamma, ln_beta):
    ids_t = input_ids.astype(jnp.int32).T.reshape(-1)
    g = _sc_gather(ids_t, token_table)
    g_t = g.reshape(LT, B, D)
    vis_t = jnp.transpose(visual_embeds, (1, 0, 2))
    pos100 = pos_table[:S]
    cls2d = class_embedding[None]
    gamma2d = ln_gamma[None]
    beta2d = ln_beta[None]
    vis_out = _tc_vis(vis_t, pos100, type_table, cls2d, gamma2d, beta2d)
    out_t = _tc_txt(vis_out, g_t, pos100, type_table, gamma2d, beta2d)
    return jnp.transpose(out_t, (1, 0, 2))


# speedup vs baseline: 3.8754x; 1.0937x over previous
import functools

import jax
import jax.numpy as jnp
from jax import lax
from jax.experimental import pallas as pl
from jax.experimental.pallas import tpu as pltpu
from jax.experimental.pallas import tpu_sc as plsc

B = 1024
LT = 50
LV = 49
D = 512
S = LV + 1 + LT

NC = 2
NS = 16
NW = NC * NS
NTOK = B * LT
PER_W = NTOK // NW
CH = 80
NCHUNK = PER_W // CH
NPAIR = NCHUNK // 2


def _sc_gather_body(ids_hbm, table_hbm, out_hbm, idx_all, buf0, buf1,
                    sg0, sg1, sw0, sw1):
    wid = lax.axis_index("s") * NC + lax.axis_index("c")
    base = wid * PER_W

    def g(i, buf, sem):
        return pltpu.make_async_copy(
            table_hbm.at[idx_all.at[pl.ds(i * CH, CH)]], buf, sem)

    def w(i, buf, sem):
        off = pl.multiple_of(base + i * CH, 8)
        return pltpu.make_async_copy(buf, out_hbm.at[pl.ds(off, CH)], sem)

    pltpu.sync_copy(ids_hbm.at[pl.ds(pl.multiple_of(base, 8), PER_W)],
                    idx_all)
    g(0, buf0, sg0).start()

    def pair(p, carry):
        i0 = p * 2
        i1 = i0 + 1

        @pl.when(p > 0)
        def _():
            w(i1 - 2, buf1, sw1).wait()

        g(i1, buf1, sg1).start()
        g(i0, buf0, sg0).wait()
        w(i0, buf0, sw0).start()

        @pl.when(p < NPAIR - 1)
        def _():
            w(i0, buf0, sw0).wait()
            g(i0 + 2, buf0, sg0).start()

        g(i1, buf1, sg1).wait()
        w(i1, buf1, sw1).start()
        return carry

    lax.fori_loop(0, NPAIR, pair, 0)
    w(NCHUNK - 2, buf0, sw0).wait()
    w(NCHUNK - 1, buf1, sw1).wait()


def _sc_gather(ids_flat, table):
    mesh = plsc.VectorSubcoreMesh(core_axis_name="c", subcore_axis_name="s")
    fn = functools.partial(
        pl.kernel,
        mesh=mesh,
        out_type=jax.ShapeDtypeStruct((NTOK, D), jnp.float32),
        scratch_types=[
            pltpu.VMEM((PER_W,), jnp.int32),
            pltpu.VMEM((CH, D), jnp.float32),
            pltpu.VMEM((CH, D), jnp.float32),
            pltpu.SemaphoreType.DMA,
            pltpu.SemaphoreType.DMA,
            pltpu.SemaphoreType.DMA,
            pltpu.SemaphoreType.DMA,
        ],
    )(_sc_gather_body)
    return fn(ids_flat, table)


BB = 32


def _ln(x, gamma, beta):
    mu = jnp.mean(x, axis=-1, keepdims=True)
    xc = x - mu
    var = jnp.mean(xc * xc, axis=-1, keepdims=True)
    return xc * lax.rsqrt(var + 1e-5) * gamma + beta


def _tc_vis_body(vis_ref, pos_ref, type_ref, cls_ref, gamma_ref, beta_ref,
                 out_ref):
    gamma = gamma_ref[...][None]
    beta = beta_ref[...][None]
    row0 = cls_ref[...] + type_ref[0:1] + pos_ref[0:1]
    y0 = _ln(row0[None], gamma, beta)
    out_ref[0:1] = jnp.broadcast_to(y0, (1, BB, D))
    bvis = (pos_ref[1:LV + 1] + type_ref[0:1])[:, None]
    out_ref[1:LV + 1] = _ln(vis_ref[...] + bvis, gamma, beta)


def _tc_txt_body(full_ref, g_ref, pos_ref, type_ref, gamma_ref, beta_ref,
                 out_ref):
    del full_ref
    gamma = gamma_ref[...][None]
    beta = beta_ref[...][None]
    btxt = (pos_ref[LV + 1:S] + type_ref[1:2])[:, None]
    out_ref[...] = _ln(g_ref[...] + btxt, gamma, beta)


def _tc_vis(vis_t, pos100, type_table, cls2d, gamma2d, beta2d):
    return pl.pallas_call(
        _tc_vis_body,
        grid=(B // BB,),
        in_specs=[
            pl.BlockSpec((LV, BB, D), lambda i: (0, i, 0)),
            pl.BlockSpec((S, D), lambda i: (0, 0)),
            pl.BlockSpec((2, D), lambda i: (0, 0)),
            pl.BlockSpec((1, D), lambda i: (0, 0)),
            pl.BlockSpec((1, D), lambda i: (0, 0)),
            pl.BlockSpec((1, D), lambda i: (0, 0)),
        ],
        out_specs=pl.BlockSpec((LV + 1, BB, D), lambda i: (0, i, 0)),
        out_shape=jax.ShapeDtypeStruct((S, B, D), jnp.float32),
    )(vis_t, pos100, type_table, cls2d, gamma2d, beta2d)


def _tc_txt(vis_out, g_t, pos100, type_table, gamma2d, beta2d):
    return pl.pallas_call(
        _tc_txt_body,
        grid=(B // BB,),
        in_specs=[
            pl.BlockSpec(memory_space=pl.ANY),
            pl.BlockSpec((LT, BB, D), lambda i: (0, i, 0)),
            pl.BlockSpec((S, D), lambda i: (0, 0)),
            pl.BlockSpec((2, D), lambda i: (0, 0)),
            pl.BlockSpec((1, D), lambda i: (0, 0)),
            pl.BlockSpec((1, D), lambda i: (0, 0)),
        ],
        out_specs=pl.BlockSpec((LT, BB, D), lambda i: (1, i, 0)),
        out_shape=jax.ShapeDtypeStruct((S, B, D), jnp.float32),
        input_output_aliases={0: 0},
    )(vis_out, g_t, pos100, type_table, gamma2d, beta2d)


def kernel(input_ids, visual_embeds, token_table, type_table,
           class_embedding, pos_table, ln_gamma, ln_beta):
    ids_t = input_ids.astype(jnp.int32).T.reshape(-1)
    g = _sc_gather(ids_t, token_table)
    g_t = g.reshape(LT, B, D)
    vis_t = jnp.transpose(visual_embeds, (1, 0, 2))
    pos100 = pos_table[:S]
    cls2d = class_embedding[None]
    gamma2d = ln_gamma[None]
    beta2d = ln_beta[None]
    vis_out = _tc_vis(vis_t, pos100, type_table, cls2d, gamma2d, beta2d)
    out_t = _tc_txt(vis_out, g_t, pos100, type_table, gamma2d, beta2d)
    return jnp.transpose(out_t, (1, 0, 2))


# speedup vs baseline: 3.9648x; 1.0231x over previous
import functools

import jax
import jax.numpy as jnp
from jax import lax
from jax.experimental import pallas as pl
from jax.experimental.pallas import tpu as pltpu
from jax.experimental.pallas import tpu_sc as plsc

B = 1024
LT = 50
LV = 49
D = 512
S = LV + 1 + LT

NC = 2
NS = 16
NW = NC * NS
NTOK = B * LT
PER_W = NTOK // NW
CH = 80
NCHUNK = PER_W // CH
NPAIR = NCHUNK // 2


def _sc_gather_body(ids_hbm, table_hbm, out_hbm, idx_all, buf0, buf1,
                    sg0, sg1, sw0, sw1):
    wid = lax.axis_index("s") * NC + lax.axis_index("c")
    base = wid * PER_W

    def g(i, buf, sem):
        return pltpu.make_async_copy(
            table_hbm.at[idx_all.at[pl.ds(i * CH, CH)]], buf, sem)

    def w(i, buf, sem):
        off = pl.multiple_of(base + i * CH, 8)
        return pltpu.make_async_copy(buf, out_hbm.at[pl.ds(off, CH)], sem)

    pltpu.sync_copy(ids_hbm.at[pl.ds(pl.multiple_of(base, 8), PER_W)],
                    idx_all)
    g(0, buf0, sg0).start()

    def pair(p, carry):
        i0 = p * 2
        i1 = i0 + 1

        @pl.when(p > 0)
        def _():
            w(i1 - 2, buf1, sw1).wait()

        g(i1, buf1, sg1).start()
        g(i0, buf0, sg0).wait()
        w(i0, buf0, sw0).start()

        @pl.when(p < NPAIR - 1)
        def _():
            w(i0, buf0, sw0).wait()
            g(i0 + 2, buf0, sg0).start()

        g(i1, buf1, sg1).wait()
        w(i1, buf1, sw1).start()
        return carry

    lax.fori_loop(0, NPAIR, pair, 0)
    w(NCHUNK - 2, buf0, sw0).wait()
    w(NCHUNK - 1, buf1, sw1).wait()


def _sc_gather(ids_flat, table):
    mesh = plsc.VectorSubcoreMesh(core_axis_name="c", subcore_axis_name="s")
    fn = functools.partial(
        pl.kernel,
        mesh=mesh,
        out_type=jax.ShapeDtypeStruct((NTOK, D), jnp.float32),
        scratch_types=[
            pltpu.VMEM((PER_W,), jnp.int32),
            pltpu.VMEM((CH, D), jnp.float32),
            pltpu.VMEM((CH, D), jnp.float32),
            pltpu.SemaphoreType.DMA,
            pltpu.SemaphoreType.DMA,
            pltpu.SemaphoreType.DMA,
            pltpu.SemaphoreType.DMA,
        ],
    )(_sc_gather_body)
    return fn(ids_flat, table)


BB = 64


def _ln(x, gamma, beta):
    mu = jnp.mean(x, axis=-1, keepdims=True)
    xc = x - mu
    var = jnp.mean(xc * xc, axis=-1, keepdims=True)
    return xc * lax.rsqrt(var + 1e-5) * gamma + beta


def _tc_vis_body(vis_ref, pos_ref, type_ref, cls_ref, gamma_ref, beta_ref,
                 out_ref):
    gamma = gamma_ref[...][None]
    beta = beta_ref[...][None]
    row0 = cls_ref[...] + type_ref[0:1] + pos_ref[0:1]
    y0 = _ln(row0[None], gamma, beta)
    out_ref[0:1] = jnp.broadcast_to(y0, (1, BB, D))
    bvis = (pos_ref[1:LV + 1] + type_ref[0:1])[:, None]
    out_ref[1:LV + 1] = _ln(vis_ref[...] + bvis, gamma, beta)


def _tc_txt_body(full_ref, g_ref, pos_ref, type_ref, gamma_ref, beta_ref,
                 out_ref):
    del full_ref
    gamma = gamma_ref[...][None]
    beta = beta_ref[...][None]
    btxt = (pos_ref[LV + 1:S] + type_ref[1:2])[:, None]
    out_ref[...] = _ln(g_ref[...] + btxt, gamma, beta)


def _tc_vis(vis_t, pos100, type_table, cls2d, gamma2d, beta2d):
    return pl.pallas_call(
        _tc_vis_body,
        grid=(B // BB,),
        in_specs=[
            pl.BlockSpec((LV, BB, D), lambda i: (0, i, 0)),
            pl.BlockSpec((S, D), lambda i: (0, 0)),
            pl.BlockSpec((2, D), lambda i: (0, 0)),
            pl.BlockSpec((1, D), lambda i: (0, 0)),
            pl.BlockSpec((1, D), lambda i: (0, 0)),
            pl.BlockSpec((1, D), lambda i: (0, 0)),
        ],
        out_specs=pl.BlockSpec((LV + 1, BB, D), lambda i: (0, i, 0)),
        out_shape=jax.ShapeDtypeStruct((S, B, D), jnp.float32),
    )(vis_t, pos100, type_table, cls2d, gamma2d, beta2d)


def _tc_txt(vis_out, g_t, pos100, type_table, gamma2d, beta2d):
    return pl.pallas_call(
        _tc_txt_body,
        grid=(B // BB,),
        in_specs=[
            pl.BlockSpec(memory_space=pl.ANY),
            pl.BlockSpec((LT, BB, D), lambda i: (0, i, 0)),
            pl.BlockSpec((S, D), lambda i: (0, 0)),
            pl.BlockSpec((2, D), lambda i: (0, 0)),
            pl.BlockSpec((1, D), lambda i: (0, 0)),
            pl.BlockSpec((1, D), lambda i: (0, 0)),
        ],
        out_specs=pl.BlockSpec((LT, BB, D), lambda i: (1, i, 0)),
        out_shape=jax.ShapeDtypeStruct((S, B, D), jnp.float32),
        input_output_aliases={0: 0},
    )(vis_out, g_t, pos100, type_table, gamma2d, beta2d)


def kernel(input_ids, visual_embeds, token_table, type_table,
           class_embedding, pos_table, ln_gamma, ln_beta):
    ids_t = input_ids.astype(jnp.int32).T.reshape(-1)
    g = _sc_gather(ids_t, token_table)
    g_t = g.reshape(LT, B, D)
    vis_t = jnp.transpose(visual_embeds, (1, 0, 2))
    pos100 = pos_table[:S]
    cls2d = class_embedding[None]
    gamma2d = ln_gamma[None]
    beta2d = ln_beta[None]
    vis_out = _tc_vis(vis_t, pos100, type_table, cls2d, gamma2d, beta2d)
    out_t = _tc_txt(vis_out, g_t, pos100, type_table, gamma2d, beta2d)
    return jnp.transpose(out_t, (1, 0, 2))
